# Initial kernel scaffold; baseline (speedup 1.0000x reference)
import jax
import jax.numpy as jnp
from jax.experimental import pallas as pl


def kernel(x, edge_index, Wq1, bq1, Wk1, bk1, Wv1, bv1, Ws1, bs1, Wq2, bq2, Wk2, bk2, Wv2, bv2, Ws2, bs2):
    raise NotImplementedError("write your pallas kernel here")


# speedup vs baseline: 3.4336x; 3.4336x over previous
import functools
import math

import jax
import jax.numpy as jnp
from jax import lax
from jax.experimental import pallas as pl
from jax.experimental.pallas import tpu as pltpu
from jax.experimental.pallas import tpu_sc as plsc

N = 10000
E = 160000
D_IN = 128
H1 = 400
HP = 416
HH = 208
HV = 448
HQ = 112
NC = 2
NS = 16
NW = NC * NS
CH = 40
CHP = 48
EPW = E // NW
NCH_W = EPW // CH
EPS = E // NS
NCH_S = EPS // CH
RPT = N // NS
RB = 1000
f32 = jnp.float32
i32 = jnp.int32


def _proj1_body(x_ref, wq, bq, wk, bk,
                wv0, bv0, wv1, bv1, wv2, bv2, wv3, bv3,
                ws0, bs0, ws1, bs1, ws2, bs2, ws3, bs3,
                q_o, k_o, v0_o, v1_o, v2_o, v3_o, s0_o, s1_o, s2_o, s3_o):
    xb = x_ref[...]
    mm = lambda w: lax.dot_general(xb, w[...], (((1,), (0,)), ((), ())),
                                   preferred_element_type=f32)
    q_o[...] = mm(wq) + bq[...]
    k_o[...] = mm(wk) + bk[...]
    v0_o[...] = mm(wv0) + bv0[...]
    v1_o[...] = mm(wv1) + bv1[...]
    v2_o[...] = mm(wv2) + bv2[...]
    v3_o[...] = mm(wv3) + bv3[...]
    s0_o[...] = mm(ws0) + bs0[...]
    s1_o[...] = mm(ws1) + bs1[...]
    s2_o[...] = mm(ws2) + bs2[...]
    s3_o[...] = mm(ws3) + bs3[...]


def _b1_body(q_hbm, k_hbm, src_hbm, dst_hbm, ex_hbm, den_hbm,
             qbuf, kbuf, src40, dst40, dst48, dot48, den, sem_q, sem_k):
    c = lax.axis_index("c")
    s = lax.axis_index("s")
    wid = s * NC + c
    zeros16 = jnp.zeros((16,), f32)

    def zden(i, carry):
        den[pl.ds(i * 16, 16)] = zeros16
        return carry
    lax.fori_loop(0, N // 16, zden, 0)
    for g in range(CHP // 16):
        dst48[pl.ds(g * 16, 16)] = jnp.zeros((16,), i32)
        dot48[pl.ds(g * 16, 16)] = zeros16

    lanes = lax.iota(i32, 16)
    inv = f32(1.0 / math.sqrt(H1))

    def chunk(j, carry):
        off = wid * EPW + j * CH
        pltpu.sync_copy(src_hbm.at[pl.ds(off, CH)], src40)
        pltpu.sync_copy(dst_hbm.at[pl.ds(off, CH)], dst40)
        pltpu.sync_copy(dst_hbm.at[pl.ds(off, CH)], dst48.at[pl.ds(0, CH)])
        cq = pltpu.async_copy(q_hbm.at[dst40], qbuf, sem_q)
        ck = pltpu.async_copy(k_hbm.at[src40], kbuf, sem_k)
        cq.wait()
        ck.wait()

        def edge(e, ecarry):
            acc = jnp.zeros((16,), f32)
            for t in range(HP // 16):
                acc = acc + qbuf[e, pl.ds(t * 16, 16)] * kbuf[e, pl.ds(t * 16, 16)]
            dsc = jnp.sum(acc)
            base = (e // 16) * 16
            dotv = dot48[pl.ds(base, 16)]
            dot48[pl.ds(base, 16)] = jnp.where(lanes == e - base,
                                               jnp.full((16,), dsc, f32), dotv)
            return ecarry
        lax.fori_loop(0, CH, edge, 0)

        for g in range(CHP // 16):
            exv = jnp.exp(dot48[pl.ds(g * 16, 16)] * inv)
            dot48[pl.ds(g * 16, 16)] = exv
            m = (lanes + g * 16) < CH
            idx = dst48[pl.ds(g * 16, 16)]
            plsc.addupdate_scatter(den, [idx], exv, mask=m)
        pltpu.sync_copy(dot48.at[pl.ds(0, CH)], ex_hbm.at[pl.ds(off, CH)])
        return carry
    lax.fori_loop(0, NCH_W, chunk, 0)
    pltpu.sync_copy(den, den_hbm.at[wid])


def _b2_body(v0_hbm, v1_hbm, v2_hbm, v3_hbm, src_hbm, dst_hbm, ex_hbm,
             num0_hbm, num1_hbm, num2_hbm, num3_hbm,
             vbuf, msg, src40, dst40, ex40, zbuf, acc_spm, sem_v):
    c = lax.axis_index("c")
    s = lax.axis_index("s")
    zeros16 = jnp.zeros((16,), f32)

    def zrow(i, carry):
        for t in range(HQ // 16):
            zbuf[i, pl.ds(t * 16, 16)] = zeros16
        return carry
    lax.fori_loop(0, 125, zrow, 0)

    def one_pass(v_hbm, num_hbm):
        def chunk(j, carry):
            off = s * EPS + j * CH
            pltpu.sync_copy(src_hbm.at[pl.ds(off, CH)], src40)
            pltpu.sync_copy(dst_hbm.at[pl.ds(off, CH)], dst40)
            pltpu.sync_copy(ex_hbm.at[pl.ds(off, CH)], ex40.at[pl.ds(0, CH)])
            pltpu.async_copy(v_hbm.at[src40], vbuf, sem_v).wait()

            def edge(e, ecarry):
                exs = ex40[pl.ds(e, 16)][0]
                for t in range(HQ // 16):
                    msg[e, pl.ds(t * 16, 16)] = vbuf[e, pl.ds(t * 16, 16)] * exs
                return ecarry
            lax.fori_loop(0, CH, edge, 0)
            pltpu.sync_copy(msg, acc_spm.at[dst40], add=True)
            return carry
        lax.fori_loop(0, NCH_S, chunk, 0)

    def copy_out(num_hbm):
        pltpu.sync_copy(acc_spm.at[pl.ds(s * RPT, RPT)],
                        num_hbm.at[pl.ds(s * RPT, RPT)])

    vq = (v0_hbm, v1_hbm, v2_hbm, v3_hbm)
    nq = (num0_hbm, num1_hbm, num2_hbm, num3_hbm)
    for p in range(2):
        for i in range(5):
            pltpu.sync_copy(zbuf, acc_spm.at[pl.ds(s * RPT + i * 125, 125)])
        plsc.subcore_barrier()

        @pl.when(c == 0)
        def _():
            one_pass(vq[p], nq[p])

        @pl.when(c == 1)
        def _():
            one_pass(vq[2 + p], nq[2 + p])
        plsc.subcore_barrier()

        @pl.when(c == 0)
        def _():
            copy_out(nq[p])

        @pl.when(c == 1)
        def _():
            copy_out(nq[2 + p])
        plsc.subcore_barrier()


def _mid_body(num0, num1, num2, num3, denp, s0, s1, s2, s3,
              wqa0, wqa1, wqa2, wqa3, wkv0, wkv1, wkv2, wkv3, bqa, bkv,
              qa_o, kv_o):
    d = jnp.sum(denp[...], axis=(0, 1))[:, None] + f32(1e-16)
    mm = lambda a, w: lax.dot_general(a, w[...], (((1,), (0,)), ((), ())),
                                      preferred_element_type=f32)
    nums = (num0, num1, num2, num3)
    sks = (s0, s1, s2, s3)
    wqas = (wqa0, wqa1, wqa2, wqa3)
    wkvs = (wkv0, wkv1, wkv2, wkv3)
    qa = bqa[...] * jnp.ones((RB, 1), f32)
    kv = bkv[...] * jnp.ones((RB, 1), f32)
    for qd in range(4):
        h = jnp.maximum(nums[qd][...] / d + sks[qd][...], 0.0)
        qa = qa + mm(h, wqas[qd])
        kv = kv + mm(h, wkvs[qd])
    qa_o[...] = qa
    kv_o[...] = kv


def _b3_body(qa_hbm, kv_hbm, src_hbm, dst_hbm, num2_hbm, den2_hbm,
             qab, kvb, msg, src40, dst40, dst48, dot48, den, zb, spm,
             sem_a, sem_b):
    c = lax.axis_index("c")
    s = lax.axis_index("s")
    wid = s * NC + c
    zeros16 = jnp.zeros((16,), f32)

    def zden(i, carry):
        den[pl.ds(i * 16, 16)] = zeros16
        return carry
    lax.fori_loop(0, N // 16, zden, 0)

    def zrow(i, carry):
        zb[i, pl.ds(0, 16)] = zeros16
        return carry
    lax.fori_loop(0, RPT, zrow, 0)
    pltpu.sync_copy(zb, spm.at[pl.ds(s * RPT, RPT)])
    for g in range(CHP // 16):
        dst48[pl.ds(g * 16, 16)] = jnp.zeros((16,), i32)
        dot48[pl.ds(g * 16, 16)] = zeros16
    plsc.subcore_barrier()

    lanes = lax.iota(i32, 16)
    half = f32(0.5)

    def chunk(j, carry):
        off = wid * EPW + j * CH
        pltpu.sync_copy(src_hbm.at[pl.ds(off, CH)], src40)
        pltpu.sync_copy(dst_hbm.at[pl.ds(off, CH)], dst40)
        pltpu.sync_copy(dst_hbm.at[pl.ds(off, CH)], dst48.at[pl.ds(0, CH)])
        ca = pltpu.async_copy(qa_hbm.at[dst40], qab, sem_a)
        cb = pltpu.async_copy(kv_hbm.at[src40], kvb, sem_b)
        ca.wait()
        cb.wait()

        def edge(e, ecarry):
            a = qab[e, pl.ds(0, 16)]
            b = kvb[e, pl.ds(0, 16)]
            dsc = jnp.sum(a * b)
            base = (e // 16) * 16
            dotv = dot48[pl.ds(base, 16)]
            dot48[pl.ds(base, 16)] = jnp.where(lanes == e - base,
                                               jnp.full((16,), dsc, f32), dotv)
            return ecarry
        lax.fori_loop(0, CH, edge, 0)

        for g in range(CHP // 16):
            exv = jnp.exp(dot48[pl.ds(g * 16, 16)] * half)
            dot48[pl.ds(g * 16, 16)] = exv
            m = (lanes + g * 16) < CH
            idx = dst48[pl.ds(g * 16, 16)]
            plsc.addupdate_scatter(den, [idx], exv, mask=m)

        def edge2(e, ecarry):
            exs = dot48[pl.ds(e, 16)][0]
            msg[e, pl.ds(0, 16)] = kvb[e, pl.ds(0, 16)] * exs
            return ecarry
        lax.fori_loop(0, CH, edge2, 0)
        pltpu.sync_copy(msg, spm.at[dst40], add=True)
        return carry
    lax.fori_loop(0, NCH_W, chunk, 0)
    plsc.subcore_barrier()
    pltpu.sync_copy(spm.at[pl.ds(s * RPT, RPT)],
                    num2_hbm.at[c, pl.ds(s * RPT, RPT)])
    pltpu.sync_copy(den, den2_hbm.at[wid])


def _fin_body(num2, den2, kv, selv, sels, out):
    d = jnp.sum(den2[...], axis=(0, 1))[:, None] + f32(1e-16)
    att = jnp.sum(num2[...], axis=0)
    mm = lambda a, w: lax.dot_general(a, w[...], (((1,), (0,)), ((), ())),
                                      preferred_element_type=f32)
    v = mm(att, selv)
    sk = mm(kv[...], sels)
    out[...] = jnp.maximum(v / d + sk, 0.0)


def kernel(x, edge_index, Wq1, bq1, Wk1, bk1, Wv1, bv1, Ws1, bs1,
           Wq2, bq2, Wk2, bk2, Wv2, bv2, Ws2, bs2):
    src = edge_index[0].astype(i32)
    dst = edge_index[1].astype(i32)

    padw = lambda w, hp: jnp.pad(w.astype(f32), ((0, 0), (0, hp - H1)))
    padb = lambda b, hp: jnp.pad(b.astype(f32), (0, hp - H1))[None, :]
    wq, bq = padw(Wq1, HP), padb(bq1, HP)
    wk, bk = padw(Wk1, HP), padb(bk1, HP)
    wv, bv = padw(Wv1, HV), padb(bv1, HV)
    ws, bs = padw(Ws1, HV), padb(bs1, HV)
    wvq = [wv[:, i * HQ:(i + 1) * HQ] for i in range(4)]
    bvq = [bv[:, i * HQ:(i + 1) * HQ] for i in range(4)]
    wsq = [ws[:, i * HQ:(i + 1) * HQ] for i in range(4)]
    bsq = [bs[:, i * HQ:(i + 1) * HQ] for i in range(4)]

    full = lambda shp: pl.BlockSpec(shp, lambda i: (0,) * len(shp))
    row = lambda shp: pl.BlockSpec(shp, lambda i: (i,) + (0,) * (len(shp) - 1))
    sds = jax.ShapeDtypeStruct

    a_ins = [x, wq, bq, wk, bk]
    a_specs = [row((RB, D_IN)), full((D_IN, HP)), full((1, HP)),
               full((D_IN, HP)), full((1, HP))]
    for i in range(4):
        a_ins += [wvq[i], bvq[i]]
        a_specs += [full((D_IN, HQ)), full((1, HQ))]
    for i in range(4):
        a_ins += [wsq[i], bsq[i]]
        a_specs += [full((D_IN, HQ)), full((1, HQ))]

    outs = pl.pallas_call(
        _proj1_body,
        grid=(N // RB,),
        in_specs=a_specs,
        out_specs=[row((RB, HP)), row((RB, HP))] + [row((RB, HQ))] * 8,
        out_shape=[sds((N, HP), f32), sds((N, HP), f32)] +
                  [sds((N, HQ), f32)] * 8,
    )(*a_ins)
    q1, k1 = outs[0], outs[1]
    v1q = outs[2:6]
    s1q = outs[6:10]

    mesh = plsc.VectorSubcoreMesh(core_axis_name="c", subcore_axis_name="s")
    sc_params = pltpu.CompilerParams(use_tc_tiling_on_sc=False,
                                     needs_layout_passes=False)

    b1 = functools.partial(
        pl.kernel,
        out_type=[sds((E,), f32), sds((NW, N), f32)],
        mesh=mesh,
        compiler_params=sc_params,
        scratch_types=[pltpu.VMEM((CH, HP), f32), pltpu.VMEM((CH, HP), f32),
                       pltpu.VMEM((CH,), i32), pltpu.VMEM((CH,), i32),
                       pltpu.VMEM((CHP,), i32), pltpu.VMEM((CHP,), f32),
                       pltpu.VMEM((N,), f32),
                       pltpu.SemaphoreType.DMA, pltpu.SemaphoreType.DMA],
    )(_b1_body)
    ex, denp = b1(q1, k1, src, dst)

    b2 = functools.partial(
        pl.kernel,
        out_type=[sds((N, HQ), f32)] * 4,
        mesh=mesh,
        compiler_params=sc_params,
        scratch_types=[pltpu.VMEM((CH, HQ), f32), pltpu.VMEM((CH, HQ), f32),
                       pltpu.VMEM((CH,), i32), pltpu.VMEM((CH,), i32),
                       pltpu.VMEM((CH + 16,), f32), pltpu.VMEM((125, HQ), f32),
                       pltpu.VMEM_SHARED((N, HQ), f32),
                       pltpu.SemaphoreType.DMA],
    )(_b2_body)
    num1q = b2(v1q[0], v1q[1], v1q[2], v1q[3], src, dst, ex)

    wqa = jnp.zeros((HV, 16), f32).at[:H1, 0:4].set(Wq2.astype(f32))
    wkv = (jnp.zeros((HV, 16), f32)
           .at[:H1, 0:4].set(Wk2.astype(f32))
           .at[:H1, 4:8].set(Wv2.astype(f32))
           .at[:H1, 8:12].set(Ws2.astype(f32)))
    bqa = jnp.zeros((1, 16), f32).at[0, 0:4].set(bq2.astype(f32))
    bkv = (jnp.zeros((1, 16), f32)
           .at[0, 0:4].set(bk2.astype(f32))
           .at[0, 4:8].set(bv2.astype(f32))
           .at[0, 8:12].set(bs2.astype(f32)))

    denp3 = denp.reshape(NW, N // RB, RB).transpose(1, 0, 2)
    c_ins = list(num1q) + [denp3] + list(s1q) + \
        [wqa[i * HQ:(i + 1) * HQ] for i in range(4)] + \
        [wkv[i * HQ:(i + 1) * HQ] for i in range(4)] + [bqa, bkv]
    c_specs = [row((RB, HQ))] * 4 + \
        [pl.BlockSpec((1, NW, RB), lambda i: (i, 0, 0))] + \
        [row((RB, HQ))] * 4 + \
        [full((HQ, 16))] * 8 + [full((1, 16))] * 2

    qa, kv = pl.pallas_call(
        _mid_body,
        grid=(N // RB,),
        in_specs=c_specs,
        out_specs=[row((RB, 16)), row((RB, 16))],
        out_shape=[sds((N, 16), f32), sds((N, 16), f32)],
    )(*c_ins)

    b3 = functools.partial(
        pl.kernel,
        out_type=[sds((NC, N, 16), f32), sds((NW, N), f32)],
        mesh=mesh,
        compiler_params=sc_params,
        scratch_types=[pltpu.VMEM((CH, 16), f32), pltpu.VMEM((CH, 16), f32),
                       pltpu.VMEM((CH, 16), f32),
                       pltpu.VMEM((CH,), i32), pltpu.VMEM((CH,), i32),
                       pltpu.VMEM((CHP,), i32), pltpu.VMEM((CHP + 16,), f32),
                       pltpu.VMEM((N,), f32), pltpu.VMEM((RPT, 16), f32),
                       pltpu.VMEM_SHARED((N, 16), f32),
                       pltpu.SemaphoreType.DMA, pltpu.SemaphoreType.DMA],
    )(_b3_body)
    num2, den2 = b3(qa, kv, src, dst)

    selv = jnp.zeros((16, 4), f32).at[4:8].set(jnp.eye(4, dtype=f32))
    sels = jnp.zeros((16, 4), f32).at[8:12].set(jnp.eye(4, dtype=f32))

    den23 = den2.reshape(NW, N // RB, RB).transpose(1, 0, 2)
    out = pl.pallas_call(
        _fin_body,
        grid=(N // RB,),
        in_specs=[pl.BlockSpec((NC, RB, 16), lambda i: (0, i, 0)),
                  pl.BlockSpec((1, NW, RB), lambda i: (i, 0, 0)),
                  row((RB, 16)), full((16, 4)), full((16, 4))],
        out_specs=row((RB, 4)),
        out_shape=sds((N, 4), f32),
    )(num2, den23, kv, selv, sels)
    return out
